# baseline (speedup 1.0000x reference)
import jax
import jax.numpy as jnp
from jax import lax
from jax.experimental import pallas as pl
from jax.experimental.pallas import tpu as pltpu
from jax.experimental.pallas import tpu_sc as plsc

_N = 10000
_NP = 10240
_E = 320000
_D = 64
_DH = _D // 2
_CHUNK = 128
_NC = 2
_NS = 16
_NW = _NC * _NS
_KPT = 80
_EPT = _KPT * _CHUNK
_EP = _EPT * _NW
_KPC = _EP // _CHUNK // _NS
_RPT = _NP // _NS
_NB = 4

_mesh = plsc.VectorSubcoreMesh(core_axis_name="c", subcore_axis_name="s")
_sc_params = pltpu.CompilerParams(use_tc_tiling_on_sc=False)


def _deg_body(dst_hbm, zvec_hbm, deg_hbm, idx_d, ones_v, deg_sp, isem, dsem):
    c = lax.axis_index("c")
    s = lax.axis_index("s")
    w = c * _NS + s
    pltpu.async_copy(dst_hbm.at[pl.ds(w * _KPT, _KPT)], idx_d, isem)
    for i in range(_CHUNK // 16):
        ones_v[pl.ds(i * 16, 16)] = jnp.full((16,), 1.0, jnp.float32)
    pltpu.sync_copy(zvec_hbm, deg_sp.at[pl.ds(s * _RPT, _RPT)])
    pltpu.make_async_copy(dst_hbm.at[pl.ds(w * _KPT, _KPT)], idx_d, isem).wait()
    plsc.subcore_barrier()

    def fire(i, carry):
        for b in range(_NB):
            pltpu.async_copy(ones_v, deg_sp.at[idx_d.at[i * _NB + b]], dsem,
                             add=True)
        return carry

    lax.fori_loop(0, _KPT // _NB, fire, 0)

    def drain(i, carry):
        for b in range(_NB):
            pltpu.make_async_copy(ones_v, deg_sp.at[idx_d.at[0]], dsem).wait()
        return carry

    lax.fori_loop(0, _KPT // _NB, drain, 0)
    plsc.subcore_barrier()
    pltpu.sync_copy(deg_sp.at[pl.ds(s * _RPT, _RPT)],
                    deg_hbm.at[c, pl.ds(s * _RPT, _RPT)])


_deg_call = pl.kernel(
    _deg_body,
    out_type=jax.ShapeDtypeStruct((_NC, _NP), jnp.float32),
    scratch_types=[
        pltpu.VMEM((_KPT, _CHUNK), jnp.int32),
        pltpu.VMEM((_CHUNK,), jnp.float32),
        pltpu.VMEM_SHARED((_NP,), jnp.float32),
        pltpu.SemaphoreType.DMA,
        pltpu.SemaphoreType.DMA,
    ],
    mesh=_mesh,
    compiler_params=_sc_params,
)


def _mp_body(y_hbm, src_hbm, dst_hbm, zrow_hbm, z_hbm, idx_s, idx_d, rows,
             y_sp, z_sp, is0, is1, gs0, gs1, gs2, gs3, ss0, ss1, ss2, ss3):
    gsem = (gs0, gs1, gs2, gs3)
    ssem = (ss0, ss1, ss2, ss3)
    c = lax.axis_index("c")
    s = lax.axis_index("s")
    col0 = c * _DH
    pltpu.async_copy(src_hbm.at[pl.ds(s * _KPC, _KPC)], idx_s, is0)
    pltpu.async_copy(dst_hbm.at[pl.ds(s * _KPC, _KPC)], idx_d, is1)
    pltpu.sync_copy(y_hbm.at[pl.ds(s * _RPT, _RPT), pl.ds(col0, _DH)],
                    y_sp.at[pl.ds(s * _RPT, _RPT)])
    pltpu.sync_copy(zrow_hbm, z_sp.at[pl.ds(s * _RPT, _RPT)])
    pltpu.make_async_copy(src_hbm.at[pl.ds(s * _KPC, _KPC)], idx_s, is0).wait()
    pltpu.make_async_copy(dst_hbm.at[pl.ds(s * _KPC, _KPC)], idx_d, is1).wait()
    plsc.subcore_barrier()

    for b in range(_NB):
        pltpu.async_copy(y_sp.at[idx_s.at[b]], rows.at[b], gsem[b])

    def group(i, carry):
        g = i * _NB
        for b in range(_NB):
            pltpu.make_async_copy(y_sp.at[idx_s.at[0]], rows.at[b],
                                  gsem[b]).wait()
            pltpu.async_copy(rows.at[b], z_sp.at[idx_d.at[g + b]], ssem[b],
                             add=True)
        for b in range(_NB):
            pltpu.make_async_copy(rows.at[b], z_sp.at[idx_d.at[0]],
                                  ssem[b]).wait()
            pltpu.async_copy(y_sp.at[idx_s.at[g + _NB + b]], rows.at[b],
                             gsem[b])
        return carry

    lax.fori_loop(0, _KPC // _NB - 1, group, 0)
    g0 = _KPC - _NB
    for b in range(_NB):
        pltpu.make_async_copy(y_sp.at[idx_s.at[0]], rows.at[b],
                              gsem[b]).wait()
        pltpu.async_copy(rows.at[b], z_sp.at[idx_d.at[g0 + b]], ssem[b],
                         add=True)
    for b in range(_NB):
        pltpu.make_async_copy(rows.at[b], z_sp.at[idx_d.at[0]],
                              ssem[b]).wait()
    plsc.subcore_barrier()
    pltpu.sync_copy(z_sp.at[pl.ds(s * _RPT, _RPT)],
                    z_hbm.at[pl.ds(s * _RPT, _RPT), pl.ds(col0, _DH)])


_mp_call = pl.kernel(
    _mp_body,
    out_type=jax.ShapeDtypeStruct((_NP, _D), jnp.float32),
    scratch_types=[
        pltpu.VMEM((_KPC, _CHUNK), jnp.int32),
        pltpu.VMEM((_KPC, _CHUNK), jnp.int32),
        pltpu.VMEM((_NB, _CHUNK, _DH), jnp.float32),
        pltpu.VMEM_SHARED((_NP, _DH), jnp.float32),
        pltpu.VMEM_SHARED((_NP, _DH), jnp.float32),
        pltpu.SemaphoreType.DMA,
        pltpu.SemaphoreType.DMA,
        pltpu.SemaphoreType.DMA,
        pltpu.SemaphoreType.DMA,
        pltpu.SemaphoreType.DMA,
        pltpu.SemaphoreType.DMA,
        pltpu.SemaphoreType.DMA,
        pltpu.SemaphoreType.DMA,
        pltpu.SemaphoreType.DMA,
        pltpu.SemaphoreType.DMA,
    ],
    mesh=_mesh,
    compiler_params=_sc_params,
)


def _lin1_body(x_ref, w1_ref, deg_ref, y_ref):
    dinv = lax.rsqrt(deg_ref[0] + deg_ref[1] + 1.0)
    xw = jnp.dot(x_ref[...], w1_ref[...],
                 preferred_element_type=jnp.float32,
                 precision=lax.Precision.HIGHEST)
    y_ref[:_N, :] = xw * dinv[:_N]
    y_ref[_N:, :] = jnp.zeros((_NP - _N, _D), jnp.float32)


def _mid_body(z_ref, y1_ref, deg_ref, b1_ref, w2_ref, y2_ref):
    dinv = lax.rsqrt(deg_ref[0] + deg_ref[1] + 1.0)
    t = (z_ref[...] + y1_ref[...]) * dinv + b1_ref[...]
    h = jnp.maximum(t, 0.0)
    rows = lax.broadcasted_iota(jnp.int32, (_NP, 1), 0)
    h = jnp.where(rows < _N, h, 0.0)
    y2_ref[...] = jnp.dot(h, w2_ref[...],
                          preferred_element_type=jnp.float32,
                          precision=lax.Precision.HIGHEST) * dinv


def _out_body(w_ref, y2_ref, deg_ref, b2_ref, out_ref):
    dinv = lax.rsqrt(deg_ref[0] + deg_ref[1] + 1.0)
    o = (w_ref[...] + y2_ref[...]) * dinv + b2_ref[...]
    out_ref[...] = o[:_N, :]


def kernel(x, edge_index, W1, b1, W2, b2):
    src = edge_index[0]
    dst = edge_index[1]
    pad_e = _EP - _E
    srcp = jnp.concatenate([src, jnp.full((pad_e,), _N, src.dtype)])
    dstp = jnp.concatenate([dst, jnp.full((pad_e,), _N, dst.dtype)])
    src2 = srcp.reshape(_EP // _CHUNK, _CHUNK)
    dst2 = dstp.reshape(_EP // _CHUNK, _CHUNK)
    zvec = jnp.zeros((_RPT,), jnp.float32)
    zrow = jnp.zeros((_RPT, _DH), jnp.float32)

    degp = _deg_call(dst2, zvec)
    deg3 = degp[:, :, None]

    y1 = pl.pallas_call(
        _lin1_body,
        out_shape=jax.ShapeDtypeStruct((_NP, _D), jnp.float32),
    )(x, W1, deg3)

    z1 = _mp_call(y1, src2, dst2, zrow)

    y2 = pl.pallas_call(
        _mid_body,
        out_shape=jax.ShapeDtypeStruct((_NP, _D), jnp.float32),
    )(z1, y1, deg3, b1[None, :], W2)

    z2 = _mp_call(y2, src2, dst2, zrow)

    out = pl.pallas_call(
        _out_body,
        out_shape=jax.ShapeDtypeStruct((_N, _D), jnp.float32),
    )(z2, y2, deg3, b2[None, :])
    return out


# speedup vs baseline: 1.1117x; 1.1117x over previous
import jax
import jax.numpy as jnp
from jax import lax
from jax.experimental import pallas as pl
from jax.experimental.pallas import tpu as pltpu
from jax.experimental.pallas import tpu_sc as plsc

_N = 10000
_NP = 10240
_E = 320000
_D = 64
_DH = _D // 2
_CHUNK = 128
_NC = 2
_NS = 16
_NW = _NC * _NS
_KPT = 80
_EPT = _KPT * _CHUNK
_EP = _EPT * _NW
_KPC = _EP // _CHUNK // _NS
_RPT = _NP // _NS
_NB = 4

_mesh = plsc.VectorSubcoreMesh(core_axis_name="c", subcore_axis_name="s")
_sc_params = pltpu.CompilerParams(use_tc_tiling_on_sc=False)


def _deg_body(dst_hbm, zvec_hbm, deg_hbm, idx_d, ones_v, deg_sp, isem, dsem):
    c = lax.axis_index("c")
    s = lax.axis_index("s")
    w = c * _NS + s
    pltpu.async_copy(dst_hbm.at[pl.ds(w * _KPT, _KPT)], idx_d, isem)
    for i in range(_CHUNK // 16):
        ones_v[pl.ds(i * 16, 16)] = jnp.full((16,), 1.0, jnp.float32)
    pltpu.sync_copy(zvec_hbm, deg_sp.at[pl.ds(s * _RPT, _RPT)])
    pltpu.make_async_copy(dst_hbm.at[pl.ds(w * _KPT, _KPT)], idx_d, isem).wait()
    plsc.subcore_barrier()

    def fire(i, carry):
        for b in range(_NB):
            pltpu.async_copy(ones_v, deg_sp.at[idx_d.at[i * _NB + b]], dsem,
                             add=True)
        return carry

    lax.fori_loop(0, _KPT // _NB, fire, 0)

    def drain(i, carry):
        for b in range(_NB):
            pltpu.make_async_copy(ones_v, deg_sp.at[idx_d.at[0]], dsem).wait()
        return carry

    lax.fori_loop(0, _KPT // _NB, drain, 0)
    plsc.subcore_barrier()
    pltpu.sync_copy(deg_sp.at[pl.ds(s * _RPT, _RPT)],
                    deg_hbm.at[c, pl.ds(s * _RPT, _RPT)])


_deg_call = pl.kernel(
    _deg_body,
    out_type=jax.ShapeDtypeStruct((_NC, _NP), jnp.float32),
    scratch_types=[
        pltpu.VMEM((_KPT, _CHUNK), jnp.int32),
        pltpu.VMEM((_CHUNK,), jnp.float32),
        pltpu.VMEM_SHARED((_NP,), jnp.float32),
        pltpu.SemaphoreType.DMA,
        pltpu.SemaphoreType.DMA,
    ],
    mesh=_mesh,
    compiler_params=_sc_params,
)


def _mp_body(y_hbm, src_hbm, dst_hbm, zrow_hbm, z_hbm, idx_s, idx_d, rows,
             y_sp, z_sp, is0, is1, gs0, gs1, gs2, gs3):
    gsem = (gs0, gs1, gs2, gs3)
    c = lax.axis_index("c")
    s = lax.axis_index("s")
    col0 = c * _DH
    pltpu.async_copy(src_hbm.at[pl.ds(s * _KPC, _KPC)], idx_s, is0)
    pltpu.async_copy(dst_hbm.at[pl.ds(s * _KPC, _KPC)], idx_d, is1)
    pltpu.sync_copy(y_hbm.at[pl.ds(s * _RPT, _RPT), pl.ds(col0, _DH)],
                    y_sp.at[pl.ds(s * _RPT, _RPT)])
    pltpu.sync_copy(zrow_hbm, z_sp.at[pl.ds(s * _RPT, _RPT)])
    pltpu.make_async_copy(src_hbm.at[pl.ds(s * _KPC, _KPC)], idx_s, is0).wait()
    pltpu.make_async_copy(dst_hbm.at[pl.ds(s * _KPC, _KPC)], idx_d, is1).wait()
    plsc.subcore_barrier()

    for b in range(_NB):
        pltpu.async_copy(y_sp.at[idx_s.at[b]], rows.at[b], gsem[b])

    def group(i, carry):
        g = i * _NB
        for b in range(_NB):
            pltpu.make_async_copy(y_sp.at[idx_s.at[0]], rows.at[b],
                                  gsem[b]).wait()
            pltpu.sync_copy(rows.at[b], z_sp.at[idx_d.at[g + b]], add=True)
            pltpu.async_copy(y_sp.at[idx_s.at[g + _NB + b]], rows.at[b],
                             gsem[b])
        return carry

    lax.fori_loop(0, _KPC // _NB - 1, group, 0)
    g0 = _KPC - _NB
    for b in range(_NB):
        pltpu.make_async_copy(y_sp.at[idx_s.at[0]], rows.at[b],
                              gsem[b]).wait()
        pltpu.sync_copy(rows.at[b], z_sp.at[idx_d.at[g0 + b]], add=True)
    plsc.subcore_barrier()
    pltpu.sync_copy(z_sp.at[pl.ds(s * _RPT, _RPT)],
                    z_hbm.at[pl.ds(s * _RPT, _RPT), pl.ds(col0, _DH)])


_mp_call = pl.kernel(
    _mp_body,
    out_type=jax.ShapeDtypeStruct((_NP, _D), jnp.float32),
    scratch_types=[
        pltpu.VMEM((_KPC, _CHUNK), jnp.int32),
        pltpu.VMEM((_KPC, _CHUNK), jnp.int32),
        pltpu.VMEM((_NB, _CHUNK, _DH), jnp.float32),
        pltpu.VMEM_SHARED((_NP, _DH), jnp.float32),
        pltpu.VMEM_SHARED((_NP, _DH), jnp.float32),
        pltpu.SemaphoreType.DMA,
        pltpu.SemaphoreType.DMA,
        pltpu.SemaphoreType.DMA,
        pltpu.SemaphoreType.DMA,
        pltpu.SemaphoreType.DMA,
        pltpu.SemaphoreType.DMA,
    ],
    mesh=_mesh,
    compiler_params=_sc_params,
)


def _lin1_body(x_ref, w1_ref, deg_ref, y_ref):
    dinv = lax.rsqrt(deg_ref[0] + deg_ref[1] + 1.0)
    xw = jnp.dot(x_ref[...], w1_ref[...],
                 preferred_element_type=jnp.float32,
                 precision=lax.Precision.HIGHEST)
    y_ref[:_N, :] = xw * dinv[:_N]
    y_ref[_N:, :] = jnp.zeros((_NP - _N, _D), jnp.float32)


def _mid_body(z_ref, y1_ref, deg_ref, b1_ref, w2_ref, y2_ref):
    dinv = lax.rsqrt(deg_ref[0] + deg_ref[1] + 1.0)
    t = (z_ref[...] + y1_ref[...]) * dinv + b1_ref[...]
    h = jnp.maximum(t, 0.0)
    rows = lax.broadcasted_iota(jnp.int32, (_NP, 1), 0)
    h = jnp.where(rows < _N, h, 0.0)
    y2_ref[...] = jnp.dot(h, w2_ref[...],
                          preferred_element_type=jnp.float32,
                          precision=lax.Precision.HIGHEST) * dinv


def _out_body(w_ref, y2_ref, deg_ref, b2_ref, out_ref):
    dinv = lax.rsqrt(deg_ref[0] + deg_ref[1] + 1.0)
    o = (w_ref[...] + y2_ref[...]) * dinv + b2_ref[...]
    out_ref[...] = o[:_N, :]


def kernel(x, edge_index, W1, b1, W2, b2):
    src = edge_index[0]
    dst = edge_index[1]
    pad_e = _EP - _E
    srcp = jnp.concatenate([src, jnp.full((pad_e,), _N, src.dtype)])
    dstp = jnp.concatenate([dst, jnp.full((pad_e,), _N, dst.dtype)])
    src2 = srcp.reshape(_EP // _CHUNK, _CHUNK)
    dst2 = dstp.reshape(_EP // _CHUNK, _CHUNK)
    zvec = jnp.zeros((_RPT,), jnp.float32)
    zrow = jnp.zeros((_RPT, _DH), jnp.float32)

    degp = _deg_call(dst2, zvec)
    deg3 = degp[:, :, None]

    y1 = pl.pallas_call(
        _lin1_body,
        out_shape=jax.ShapeDtypeStruct((_NP, _D), jnp.float32),
    )(x, W1, deg3)

    z1 = _mp_call(y1, src2, dst2, zrow)

    y2 = pl.pallas_call(
        _mid_body,
        out_shape=jax.ShapeDtypeStruct((_NP, _D), jnp.float32),
    )(z1, y1, deg3, b1[None, :], W2)

    z2 = _mp_call(y2, src2, dst2, zrow)

    out = pl.pallas_call(
        _out_body,
        out_shape=jax.ShapeDtypeStruct((_N, _D), jnp.float32),
    )(z2, y2, deg3, b2[None, :])
    return out
